# baseline (speedup 1.0000x reference)
import functools

import jax
import jax.numpy as jnp
from jax import lax
from jax.experimental import pallas as pl
from jax.experimental.pallas import tpu as pltpu
from jax.experimental.pallas import tpu_sc as plsc

NN = 10000
NE = 320000
D = 128
NLAYERS = 7

NC, NS, L = 2, 16, 16
NW = NC * NS
R = 313
NN_PAD = NW * R
PK = 512
C = 64
FLUSH = 4096
STG = FLUSH + 160
EMAX = 80 * FLUSH
CHK = 2048
NCHKS = NE // CHK + 1

_mesh = plsc.VectorSubcoreMesh(core_axis_name="c", subcore_axis_name="s")


def _partition_edges(src, dst):
    @functools.partial(
        pl.kernel,
        out_type=(
            jax.ShapeDtypeStruct((NW * EMAX,), jnp.int32),
            jax.ShapeDtypeStruct((NW * L,), jnp.int32),
        ),
        mesh=_mesh,
        scratch_types=[
            pltpu.VMEM((CHK,), jnp.int32),
            pltpu.VMEM((CHK,), jnp.int32),
            pltpu.VMEM((STG,), jnp.int32),
            pltpu.VMEM((L,), jnp.int32),
            pltpu.SMEM((4,), jnp.int32),
        ],
    )
    def kern(src_hbm, dst_hbm, ep_hbm, cnt_hbm, dbuf, sbuf, stg, cvec, st):
        wid = lax.axis_index("s") * NC + lax.axis_index("c")
        lo = wid * R
        ebase = wid * EMAX
        lanes = lax.iota(jnp.int32, L)
        targets = lanes + 1
        zer = jnp.zeros((L,), jnp.int32)
        trash = jnp.full((L,), R, jnp.int32)
        st[0] = 0
        st[1] = 0

        def outer(k, _):
            pltpu.sync_copy(dst_hbm.at[pl.ds(k * CHK, CHK)], dbuf)
            pltpu.sync_copy(src_hbm.at[pl.ds(k * CHK, CHK)], sbuf)

            def inner(g, _):
                dvec = dbuf[pl.ds(g * L, L)]
                svec = sbuf[pl.ds(g * L, L)]
                dl = dvec - lo
                m = (dl >= 0) & (dl < R)
                p = jnp.where(m, 1, 0)
                for s in (1, 2, 4, 8):
                    sh = p.at[jnp.maximum(lanes - s, 0)].get(
                        mode="promise_in_bounds")
                    p = p + jnp.where(lanes >= s, sh, 0)
                cnt = p[L - 1]
                pos = zer
                for s in (8, 4, 2, 1):
                    cand = pos + s
                    pv = p.at[cand - 1].get(mode="promise_in_bounds")
                    pos = jnp.where(pv < targets, cand, pos)
                packed = svec * PK + dl
                out = packed.at[pos].get(mode="promise_in_bounds")
                out = jnp.where(lanes < cnt, out, trash)
                cur = st[0]
                stg[pl.ds(cur, L)] = out
                st[0] = cur + cnt

                @pl.when(st[0] >= FLUSH)
                def _fl():
                    hb = st[1]
                    off = pl.multiple_of(ebase + hb, FLUSH)
                    pltpu.sync_copy(stg.at[pl.ds(0, FLUSH)],
                                    ep_hbm.at[pl.ds(off, FLUSH)])
                    stg[pl.ds(0, L)] = stg[pl.ds(FLUSH, L)]
                    st[0] = st[0] - FLUSH
                    st[1] = hb + FLUSH

                return 0

            lax.fori_loop(0, CHK // L, inner, 0)
            return 0

        lax.fori_loop(0, NCHKS, outer, 0)

        cur = st[0]
        for t in range(C // L):
            stg[pl.ds(cur + t * L, L)] = trash
        cur_pad = ((cur + C - 1) // C) * C
        hb = st[1]
        off = pl.multiple_of(ebase + hb, FLUSH)
        pltpu.sync_copy(stg.at[pl.ds(0, FLUSH)],
                        ep_hbm.at[pl.ds(off, FLUSH)])
        cvec[...] = zer + (hb + cur_pad)
        pltpu.sync_copy(cvec, cnt_hbm.at[pl.ds(pl.multiple_of(wid * L, L), L)])

    return kern(src, dst)


def _segmax(h, ep, cnt):
    HW = D // 2
    RB = 32

    @functools.partial(
        pl.kernel,
        out_type=jax.ShapeDtypeStruct((NN_PAD * D,), jnp.int32),
        mesh=_mesh,
        scratch_types=[
            pltpu.VMEM_SHARED((NN // 2, D), jnp.int32),
            pltpu.VMEM(((R + 1) * HW,), jnp.int32),
            pltpu.VMEM((C,), jnp.int32),
            pltpu.VMEM((C,), jnp.int32),
            pltpu.VMEM((C,), jnp.int32),
            pltpu.VMEM((C,), jnp.int32),
            pltpu.VMEM((C, D), jnp.int32),
            pltpu.VMEM((C, D), jnp.int32),
            pltpu.VMEM((RB * D,), jnp.int32),
            pltpu.VMEM((L,), jnp.int32),
            pltpu.SemaphoreType.DMA,
            pltpu.SemaphoreType.DMA,
            pltpu.SemaphoreType.DMA,
            pltpu.SemaphoreType.DMA,
        ],
    )
    def kern(h_hbm, ep_hbm, cnt_hbm, agg_hbm,
             hsh, acc, pb0, pb1, ia0, ia1, rb0, rb1, wstg, cvec,
             sp0, sp1, sg0, sg1):
        wid = lax.axis_index("s") * NC + lax.axis_index("c")
        lo = wid * R
        ebase = wid * EMAX
        himask = jnp.full((L,), -65536, jnp.int32)
        pninf = jnp.full((L,), -8323200, jnp.int32)
        ninf = jnp.full((L,), -jnp.inf, jnp.float32)
        fzero = jnp.zeros((L,), jnp.float32)
        pbs = (pb0, pb1)
        ias = (ia0, ia1)
        rbs = (rb0, rb1)
        sps = (sp0, sp1)
        sgs = (sg0, sg1)

        sid = lax.axis_index("s")

        @pl.when(sid < NS - 1)
        def _stage():
            off = pl.multiple_of(sid * 312, 8)
            pltpu.sync_copy(h_hbm.at[pl.ds(off, 312)],
                            hsh.at[pl.ds(off, 312)])

        @pl.when(sid == NS - 1)
        def _stage_last():
            pltpu.sync_copy(h_hbm.at[pl.ds(4680, NN // 2 - 4680)],
                            hsh.at[pl.ds(4680, NN // 2 - 4680)])

        def initr(r, _):
            acc[pl.ds(r * L, L)] = pninf
            return 0

        lax.fori_loop(0, (R + 1) * HW // L, initr, 0)
        plsc.subcore_barrier()

        pltpu.sync_copy(cnt_hbm.at[pl.ds(pl.multiple_of(wid * L, L), L)], cvec)
        n = cvec[...][0]
        nch = n // C

        def load_start(k, b):
            base = pl.multiple_of(ebase + k * C, C)
            return pltpu.async_copy(ep_hbm.at[pl.ds(base, C)], pbs[b], sps[b])

        def drain_load(b):
            pltpu.make_async_copy(ep_hbm.at[pl.ds(0, C)], pbs[b], sps[b]).wait()

        def build_and_gather(b):
            pb = pbs[b]
            iref = ias[b]
            for g in range(C // L):
                v = pb[pl.ds(g * L, L)]
                iref[pl.ds(g * L, L)] = lax.shift_right_logical(v, 10)
            return pltpu.async_copy(hsh.at[iref], rbs[b], sgs[b])

        def drain_gather(b):
            pltpu.make_async_copy(hsh.at[ias[b]], rbs[b], sgs[b]).wait()

        def unpack(iv):
            lo_f = lax.bitcast_convert_type(iv << 16, jnp.float32)
            hi_f = lax.bitcast_convert_type(iv & himask, jnp.float32)
            return lo_f, hi_f

        def repack(lo_f, hi_f):
            li = lax.bitcast_convert_type(lo_f, jnp.int32)
            hi = lax.bitcast_convert_type(hi_f, jnp.int32)
            return lax.shift_right_logical(li, 16) | (hi & himask)

        def accumulate(b):
            pb = pbs[b]
            rb = rbs[b]
            nq = HW // L

            def grp(g, _):
                pv = pb[pl.ds(g * L, L)]
                av = (pv & (PK - 1)) * HW
                rv = (lax.shift_right_logical(pv, 9) & 1) * HW
                addrs = [av[e] for e in range(L)]
                raddrs = [rv[e] for e in range(L)]
                for e in range(L):
                    base = addrs[e]
                    rh = raddrs[e]
                    ei = g * L + e
                    rvs = [rb[ei, pl.ds(rh + q * L, L)] for q in range(nq)]
                    avs = [acc[pl.ds(base + q * L, L)] for q in range(nq)]
                    for q in range(nq):
                        rl, rh = unpack(rvs[q])
                        al, ah = unpack(avs[q])
                        acc[pl.ds(base + q * L, L)] = repack(
                            jnp.maximum(al, rl), jnp.maximum(ah, rh))
                return 0

            lax.fori_loop(0, C // L, grp, 0)

        @pl.when(nch > 0)
        def _pro0():
            load_start(0, 0)
            drain_load(0)
            build_and_gather(0)

        @pl.when(nch > 1)
        def _pro1():
            load_start(1, 1)

        def pair(p2, _):
            for b in (0, 1):
                k = p2 * 2 + b

                @pl.when(k < nch)
                def _step():
                    @pl.when(k + 1 < nch)
                    def _bnext():
                        drain_load(1 - b)
                        build_and_gather(1 - b)

                    drain_gather(b)
                    accumulate(b)

                    @pl.when(k + 2 < nch)
                    def _anext():
                        load_start(k + 2, b)

            return 0

        lax.fori_loop(0, (nch + 1) // 2, pair, 0)

        def wblock(blk, nrows):
            def wrow(rr, _):
                r = blk * RB + rr

                def wq(q, _):
                    iv = acc[pl.ds(r * HW + q * L, L)]
                    lo_f, hi_f = unpack(iv)
                    lo_f = jnp.where(lo_f == ninf, fzero, lo_f)
                    hi_f = jnp.where(hi_f == ninf, fzero, hi_f)
                    base = rr * D
                    wstg[pl.ds(base + q * L, L)] = lax.bitcast_convert_type(
                        lo_f, jnp.int32)
                    wstg[pl.ds(base + (HW // L + q) * L, L)] = (
                        lax.bitcast_convert_type(hi_f, jnp.int32))
                    return 0

                lax.fori_loop(0, HW // L, wq, 0)
                return 0

            lax.fori_loop(0, nrows, wrow, 0)
            dst = pl.multiple_of((lo + blk * RB) * D, D)
            pltpu.sync_copy(wstg.at[pl.ds(0, nrows * D)],
                            agg_hbm.at[pl.ds(dst, nrows * D)])

        for blk in range(R // RB):
            wblock(blk, RB)
        wblock(R // RB, R % RB)

    return kern(h, ep, cnt)


def _tc_layer(agg, h, wl, wr, bias, last):
    M = 1000
    G = NN // M

    def body(a_ref, h_ref, wl_ref, wr_ref, b_ref, o_ref):
        acc = jnp.dot(a_ref[...], wl_ref[...], preferred_element_type=jnp.float32)
        acc = acc + jnp.dot(h_ref[...], wr_ref[...], preferred_element_type=jnp.float32)
        acc = acc + b_ref[...]
        if last:
            o_ref[...] = jnp.tanh(acc) * 0.5
        else:
            o_ref[...] = jnp.where(acc >= 0, acc, 0.02 * acc)

    return pl.pallas_call(
        body,
        grid=(G,),
        in_specs=[
            pl.BlockSpec((M, D), lambda i: (i, 0)),
            pl.BlockSpec((M, D), lambda i: (i, 0)),
            pl.BlockSpec((D, D), lambda i: (0, 0)),
            pl.BlockSpec((D, D), lambda i: (0, 0)),
            pl.BlockSpec((1, D), lambda i: (0, 0)),
        ],
        out_specs=pl.BlockSpec((M, D), lambda i: (i, 0)),
        out_shape=jax.ShapeDtypeStruct((NN, D), jnp.float32),
    )(agg, h, wl, wr, bias)


def kernel(x, edge_index, Wl, Wr, b):
    src = edge_index[0].astype(jnp.int32)
    dst = edge_index[1].astype(jnp.int32)
    pad = NCHKS * CHK - NE
    src = jnp.concatenate([src, jnp.zeros((pad,), jnp.int32)])
    dst = jnp.concatenate([dst, jnp.full((pad,), NN_PAD, jnp.int32)])
    ep, cnt = _partition_edges(src, dst)
    b2 = b.reshape(NLAYERS, 1, D)
    h = x
    for i in range(NLAYERS):
        hb = lax.bitcast_convert_type(
            h.astype(jnp.bfloat16).reshape(NN // 2, 2, D // 2, 2),
            jnp.int32).reshape(NN // 2, D)
        aggp = lax.bitcast_convert_type(_segmax(hb, ep, cnt), jnp.float32)
        agg = (aggp.reshape(NN_PAD, 2, D // 2)
               .transpose(0, 2, 1).reshape(NN_PAD, D)[:NN])
        h = _tc_layer(agg, h, Wl[i], Wr[i], b2[i], last=(i == NLAYERS - 1))
    return h


# speedup vs baseline: 2.0255x; 2.0255x over previous
import functools

import jax
import jax.numpy as jnp
from jax import lax
from jax.experimental import pallas as pl
from jax.experimental.pallas import tpu as pltpu
from jax.experimental.pallas import tpu_sc as plsc

NN = 10000
NE = 320000
D = 128
NLAYERS = 7

NC, NS, L = 2, 16, 16
NW = NC * NS
R = 313
NN_PAD = NW * R
PK = 512
C = 64
FLUSH = 4096
STG = FLUSH + 160
EMAX = 80 * FLUSH
CHK = 2048
NCHKS = NE // CHK + 1

_mesh = plsc.VectorSubcoreMesh(core_axis_name="c", subcore_axis_name="s")


def _partition_edges(src, dst):
    @functools.partial(
        pl.kernel,
        out_type=(
            jax.ShapeDtypeStruct((NW * EMAX,), jnp.int32),
            jax.ShapeDtypeStruct((NW * L,), jnp.int32),
        ),
        mesh=_mesh,
        scratch_types=[
            pltpu.VMEM((CHK,), jnp.int32),
            pltpu.VMEM((CHK,), jnp.int32),
            pltpu.VMEM((STG,), jnp.int32),
            pltpu.VMEM((L,), jnp.int32),
            pltpu.SMEM((4,), jnp.int32),
        ],
    )
    def kern(src_hbm, dst_hbm, ep_hbm, cnt_hbm, dbuf, sbuf, stg, cvec, st):
        wid = lax.axis_index("s") * NC + lax.axis_index("c")
        lo = wid * R
        ebase = wid * EMAX
        lanes = lax.iota(jnp.int32, L)
        targets = lanes + 1
        zer = jnp.zeros((L,), jnp.int32)
        trash = jnp.full((L,), R, jnp.int32)
        st[0] = 0
        st[1] = 0

        def outer(k, _):
            pltpu.sync_copy(dst_hbm.at[pl.ds(k * CHK, CHK)], dbuf)
            pltpu.sync_copy(src_hbm.at[pl.ds(k * CHK, CHK)], sbuf)

            def inner(g, _):
                dvec = dbuf[pl.ds(g * L, L)]
                svec = sbuf[pl.ds(g * L, L)]
                dl = dvec - lo
                m = (dl >= 0) & (dl < R)
                p = jnp.where(m, 1, 0)
                for s in (1, 2, 4, 8):
                    sh = p.at[jnp.maximum(lanes - s, 0)].get(
                        mode="promise_in_bounds")
                    p = p + jnp.where(lanes >= s, sh, 0)
                cnt = p[L - 1]
                pos = zer
                for s in (8, 4, 2, 1):
                    cand = pos + s
                    pv = p.at[cand - 1].get(mode="promise_in_bounds")
                    pos = jnp.where(pv < targets, cand, pos)
                packed = svec * PK + dl
                out = packed.at[pos].get(mode="promise_in_bounds")
                out = jnp.where(lanes < cnt, out, trash)
                cur = st[0]
                stg[pl.ds(cur, L)] = out
                st[0] = cur + cnt

                @pl.when(st[0] >= FLUSH)
                def _fl():
                    hb = st[1]
                    off = pl.multiple_of(ebase + hb, FLUSH)
                    pltpu.sync_copy(stg.at[pl.ds(0, FLUSH)],
                                    ep_hbm.at[pl.ds(off, FLUSH)])
                    stg[pl.ds(0, L)] = stg[pl.ds(FLUSH, L)]
                    st[0] = st[0] - FLUSH
                    st[1] = hb + FLUSH

                return 0

            lax.fori_loop(0, CHK // L, inner, 0)
            return 0

        lax.fori_loop(0, NCHKS, outer, 0)

        cur = st[0]
        for t in range(C // L):
            stg[pl.ds(cur + t * L, L)] = trash
        cur_pad = ((cur + C - 1) // C) * C
        hb = st[1]
        off = pl.multiple_of(ebase + hb, FLUSH)
        pltpu.sync_copy(stg.at[pl.ds(0, FLUSH)],
                        ep_hbm.at[pl.ds(off, FLUSH)])
        cvec[...] = zer + (hb + cur_pad)
        pltpu.sync_copy(cvec, cnt_hbm.at[pl.ds(pl.multiple_of(wid * L, L), L)])

    return kern(src, dst)


def _segmax(h, ep, cnt):
    CI = 128
    CC = 256

    @functools.partial(
        pl.kernel,
        out_type=jax.ShapeDtypeStruct((NN_PAD * D,), jnp.float32),
        mesh=_mesh,
        scratch_types=[
            pltpu.VMEM(((R + 1) * D,), jnp.float32),
            pltpu.VMEM((CC,), jnp.int32),
            pltpu.VMEM((CC,), jnp.int32),
            pltpu.VMEM((CI,), jnp.int32),
            pltpu.VMEM((CI,), jnp.int32),
            pltpu.VMEM((CI,), jnp.int32),
            pltpu.VMEM((CI,), jnp.int32),
            pltpu.VMEM((CC, D), jnp.float32),
            pltpu.VMEM((CC, D), jnp.float32),
            pltpu.VMEM((L,), jnp.int32),
            pltpu.SemaphoreType.DMA,
            pltpu.SemaphoreType.DMA,
            pltpu.SemaphoreType.DMA,
            pltpu.SemaphoreType.DMA,
        ],
    )
    def kern(h_hbm, ep_hbm, cnt_hbm, agg_hbm,
             acc, pb0, pb1, ia0, ib0, ia1, ib1, rb0, rb1, cvec,
             sp0, sp1, sg0, sg1):
        wid = lax.axis_index("s") * NC + lax.axis_index("c")
        lo = wid * R
        ebase = wid * EMAX
        ninf = jnp.full((L,), -jnp.inf, jnp.float32)
        pbs = (pb0, pb1)
        ias = (ia0, ia1)
        ibs = (ib0, ib1)
        rbs = (rb0, rb1)
        sps = (sp0, sp1)
        sgs = (sg0, sg1)

        def initr(r, _):
            acc[pl.ds(r * L, L)] = ninf
            return 0

        lax.fori_loop(0, (R + 1) * D // L, initr, 0)

        pltpu.sync_copy(cnt_hbm.at[pl.ds(pl.multiple_of(wid * L, L), L)], cvec)
        n = cvec[...][0]
        nch = n // CC

        def load_start(k, b):
            base = pl.multiple_of(ebase + k * CC, C)
            return pltpu.async_copy(ep_hbm.at[pl.ds(base, CC)], pbs[b], sps[b])

        def drain_load(b):
            pltpu.make_async_copy(ep_hbm.at[pl.ds(0, CC)], pbs[b], sps[b]).wait()

        def build_and_gather(b):
            pb = pbs[b]
            for half, iref in ((0, ias[b]), (1, ibs[b])):
                for g in range(CI // L):
                    v = pb[pl.ds(half * CI + g * L, L)]
                    iref[pl.ds(g * L, L)] = lax.shift_right_logical(v, 9)
            ca = pltpu.async_copy(h_hbm.at[ias[b]], rbs[b].at[pl.ds(0, CI)],
                                  sgs[b])
            cb = pltpu.async_copy(h_hbm.at[ibs[b]], rbs[b].at[pl.ds(CI, CI)],
                                  sgs[b])
            return ca, cb

        def drain_gather(b):
            pltpu.make_async_copy(h_hbm.at[ias[b]], rbs[b].at[pl.ds(0, CI)],
                                  sgs[b]).wait()
            pltpu.make_async_copy(h_hbm.at[ibs[b]], rbs[b].at[pl.ds(CI, CI)],
                                  sgs[b]).wait()

        def accumulate(b):
            pb = pbs[b]
            rb = rbs[b]
            nj = D // L

            def grp(g, _):
                av = (pb[pl.ds(g * L, L)] & (PK - 1)) * D
                addrs = [av[e] for e in range(L)]
                for e in range(L):
                    base = addrs[e]
                    ei = g * L + e
                    rsl = [rb[ei, pl.ds(j * L, L)] for j in range(nj)]
                    asl = [acc[pl.ds(base + j * L, L)] for j in range(nj)]
                    for j in range(nj):
                        acc[pl.ds(base + j * L, L)] = jnp.maximum(asl[j], rsl[j])
                return 0

            lax.fori_loop(0, CC // L, grp, 0)

        @pl.when(nch > 0)
        def _pro0():
            load_start(0, 0)
            drain_load(0)
            build_and_gather(0)

        @pl.when(nch > 1)
        def _pro1():
            load_start(1, 1)

        def pair(p2, _):
            for b in (0, 1):
                k = p2 * 2 + b

                @pl.when(k < nch)
                def _step():
                    @pl.when(k + 1 < nch)
                    def _bnext():
                        drain_load(1 - b)
                        build_and_gather(1 - b)

                    drain_gather(b)
                    accumulate(b)

                    @pl.when(k + 2 < nch)
                    def _anext():
                        load_start(k + 2, b)

            return 0

        lax.fori_loop(0, (nch + 1) // 2, pair, 0)

        def tail(t, _):
            k = nch * CC + t * C

            @pl.when(k < n)
            def _tl():
                base = pl.multiple_of(ebase + k, C)
                pltpu.sync_copy(ep_hbm.at[pl.ds(base, C)], pb0.at[pl.ds(0, C)])
                zer = jnp.zeros((L,), jnp.int32)
                for g in range(CI // L):
                    if g < C // L:
                        v = pb0[pl.ds(g * L, L)]
                        ia0[pl.ds(g * L, L)] = lax.shift_right_logical(v, 9)
                    else:
                        ia0[pl.ds(g * L, L)] = zer
                pltpu.async_copy(h_hbm.at[ia0], rb0.at[pl.ds(0, CI)],
                                 sg0).wait()

                def grp(g, _):
                    av = (pb0[pl.ds(g * L, L)] & (PK - 1)) * D
                    addrs = [av[e] for e in range(L)]
                    for e in range(L):
                        bs = addrs[e]
                        ei = g * L + e
                        rsl = [rb0[ei, pl.ds(j * L, L)] for j in range(D // L)]
                        asl = [acc[pl.ds(bs + j * L, L)] for j in range(D // L)]
                        for j in range(D // L):
                            acc[pl.ds(bs + j * L, L)] = jnp.maximum(
                                asl[j], rsl[j])
                    return 0

                lax.fori_loop(0, C // L, grp, 0)

            return 0

        lax.fori_loop(0, CC // C, tail, 0)

        def fixr(r, _):
            sl = pl.ds(r * L, L)
            v = acc[sl]
            acc[sl] = jnp.where(v == -jnp.inf, 0.0, v)
            return 0

        lax.fori_loop(0, R * D // L, fixr, 0)
        pltpu.sync_copy(acc.at[pl.ds(0, R * D)],
                        agg_hbm.at[pl.ds(pl.multiple_of(lo * D, D), R * D)])

    return kern(h, ep, cnt)


def _tc_layer(agg, h, wl, wr, bias, last):
    M = 1000
    G = NN // M

    def body(a_ref, h_ref, wl_ref, wr_ref, b_ref, o_ref):
        acc = jnp.dot(a_ref[...], wl_ref[...], preferred_element_type=jnp.float32)
        acc = acc + jnp.dot(h_ref[...], wr_ref[...], preferred_element_type=jnp.float32)
        acc = acc + b_ref[...]
        if last:
            o_ref[...] = jnp.tanh(acc) * 0.5
        else:
            o_ref[...] = jnp.where(acc >= 0, acc, 0.02 * acc)

    return pl.pallas_call(
        body,
        grid=(G,),
        in_specs=[
            pl.BlockSpec((M, D), lambda i: (i, 0)),
            pl.BlockSpec((M, D), lambda i: (i, 0)),
            pl.BlockSpec((D, D), lambda i: (0, 0)),
            pl.BlockSpec((D, D), lambda i: (0, 0)),
            pl.BlockSpec((1, D), lambda i: (0, 0)),
        ],
        out_specs=pl.BlockSpec((M, D), lambda i: (i, 0)),
        out_shape=jax.ShapeDtypeStruct((NN, D), jnp.float32),
    )(agg, h, wl, wr, bias)


def kernel(x, edge_index, Wl, Wr, b):
    src = edge_index[0].astype(jnp.int32)
    dst = edge_index[1].astype(jnp.int32)
    pad = NCHKS * CHK - NE
    src = jnp.concatenate([src, jnp.zeros((pad,), jnp.int32)])
    dst = jnp.concatenate([dst, jnp.full((pad,), NN_PAD, jnp.int32)])
    ep, cnt = _partition_edges(src, dst)
    b2 = b.reshape(NLAYERS, 1, D)
    h = x
    for i in range(NLAYERS):
        agg = _segmax(h, ep, cnt).reshape(NN_PAD, D)[:NN]
        h = _tc_layer(agg, h, Wl[i], Wr[i], b2[i], last=(i == NLAYERS - 1))
    return h


# speedup vs baseline: 2.1718x; 1.0722x over previous
import functools

import jax
import jax.numpy as jnp
from jax import lax
from jax.experimental import pallas as pl
from jax.experimental.pallas import tpu as pltpu
from jax.experimental.pallas import tpu_sc as plsc

NN = 10000
NE = 320000
D = 128
NLAYERS = 7

NC, NS, L = 2, 16, 16
NW = NC * NS
R = 313
NN_PAD = NW * R
PK = 512
C = 64
FLUSH = 4096
STG = FLUSH + 160
EMAX = 80 * FLUSH
CHK = 2048
NCHKS = NE // CHK + 1

_mesh = plsc.VectorSubcoreMesh(core_axis_name="c", subcore_axis_name="s")


def _partition_edges(src, dst):
    @functools.partial(
        pl.kernel,
        out_type=(
            jax.ShapeDtypeStruct((NW * EMAX,), jnp.int32),
            jax.ShapeDtypeStruct((NW * L,), jnp.int32),
        ),
        mesh=_mesh,
        scratch_types=[
            pltpu.VMEM((CHK,), jnp.int32),
            pltpu.VMEM((CHK,), jnp.int32),
            pltpu.VMEM((CHK,), jnp.int32),
            pltpu.VMEM((CHK,), jnp.int32),
            pltpu.VMEM((STG,), jnp.int32),
            pltpu.VMEM((L,), jnp.int32),
            pltpu.SMEM((4,), jnp.int32),
            pltpu.SemaphoreType.DMA,
            pltpu.SemaphoreType.DMA,
        ],
    )
    def kern(src_hbm, dst_hbm, ep_hbm, cnt_hbm,
             dbuf0, dbuf1, sbuf0, sbuf1, stg, cvec, st, sl0, sl1):
        wid = lax.axis_index("s") * NC + lax.axis_index("c")
        lo = wid * R
        ebase = wid * EMAX
        lanes = lax.iota(jnp.int32, L)
        targets = lanes + 1
        zer = jnp.zeros((L,), jnp.int32)
        trash = jnp.full((L,), R, jnp.int32)
        st[0] = 0
        st[1] = 0

        dbufs = (dbuf0, dbuf1)
        sbufs = (sbuf0, sbuf1)
        sls = (sl0, sl1)

        def scan_start(k, b):
            off = pl.multiple_of(k * CHK, 8)
            pltpu.async_copy(dst_hbm.at[pl.ds(off, CHK)], dbufs[b], sls[b])
            pltpu.async_copy(src_hbm.at[pl.ds(off, CHK)], sbufs[b], sls[b])

        def scan_drain(b):
            pltpu.make_async_copy(dst_hbm.at[pl.ds(0, CHK)], dbufs[b],
                                  sls[b]).wait()
            pltpu.make_async_copy(src_hbm.at[pl.ds(0, CHK)], sbufs[b],
                                  sls[b]).wait()

        def outer(k, b):
            dbuf = dbufs[b]
            sbuf = sbufs[b]
            scan_drain(b)
            if True:

                def inner(g, _):
                    dvec = dbuf[pl.ds(g * L, L)]
                    svec = sbuf[pl.ds(g * L, L)]
                    dl = dvec - lo
                    m = (dl >= 0) & (dl < R)
                    p = jnp.where(m, 1, 0)
                    for s in (1, 2, 4, 8):
                        sh = p.at[jnp.maximum(lanes - s, 0)].get(
                            mode="promise_in_bounds")
                        p = p + jnp.where(lanes >= s, sh, 0)
                    cnt = p[L - 1]
                    pos = zer
                    for s in (8, 4, 2, 1):
                        cand = pos + s
                        pv = p.at[cand - 1].get(mode="promise_in_bounds")
                        pos = jnp.where(pv < targets, cand, pos)
                    packed = svec * PK + dl
                    out = packed.at[pos].get(mode="promise_in_bounds")
                    out = jnp.where(lanes < cnt, out, trash)
                    cur = st[0]
                    stg[pl.ds(cur, L)] = out
                    st[0] = cur + cnt

                    @pl.when(st[0] >= FLUSH)
                    def _fl():
                        hb = st[1]
                        off = pl.multiple_of(ebase + hb, FLUSH)
                        pltpu.sync_copy(stg.at[pl.ds(0, FLUSH)],
                                        ep_hbm.at[pl.ds(off, FLUSH)])
                        stg[pl.ds(0, L)] = stg[pl.ds(FLUSH, L)]
                        st[0] = st[0] - FLUSH
                        st[1] = hb + FLUSH

                    return 0

                lax.fori_loop(0, CHK // L, inner, 0)

        scan_start(0, 0)

        def opair(p2, _):
            for b in (0, 1):
                k = p2 * 2 + b

                @pl.when(k < NCHKS)
                def _ostep():
                    @pl.when(k + 1 < NCHKS)
                    def _onext():
                        scan_start(k + 1, 1 - b)

                    outer(k, b)

            return 0

        lax.fori_loop(0, (NCHKS + 1) // 2, opair, 0)

        cur = st[0]
        for t in range(C // L):
            stg[pl.ds(cur + t * L, L)] = trash
        cur_pad = ((cur + C - 1) // C) * C
        hb = st[1]
        off = pl.multiple_of(ebase + hb, FLUSH)
        pltpu.sync_copy(stg.at[pl.ds(0, FLUSH)],
                        ep_hbm.at[pl.ds(off, FLUSH)])
        cvec[...] = zer + (hb + cur_pad)
        pltpu.sync_copy(cvec, cnt_hbm.at[pl.ds(pl.multiple_of(wid * L, L), L)])

    return kern(src, dst)


def _segmax(h, ep, cnt):
    CI = 128
    CC = 256

    @functools.partial(
        pl.kernel,
        out_type=jax.ShapeDtypeStruct((NN_PAD * D,), jnp.float32),
        mesh=_mesh,
        scratch_types=[
            pltpu.VMEM(((R + 1) * D,), jnp.float32),
            pltpu.VMEM((CC,), jnp.int32),
            pltpu.VMEM((CC,), jnp.int32),
            pltpu.VMEM((CI,), jnp.int32),
            pltpu.VMEM((CI,), jnp.int32),
            pltpu.VMEM((CI,), jnp.int32),
            pltpu.VMEM((CI,), jnp.int32),
            pltpu.VMEM((CC, D), jnp.float32),
            pltpu.VMEM((CC, D), jnp.float32),
            pltpu.VMEM((L,), jnp.int32),
            pltpu.SemaphoreType.DMA,
            pltpu.SemaphoreType.DMA,
            pltpu.SemaphoreType.DMA,
            pltpu.SemaphoreType.DMA,
        ],
    )
    def kern(h_hbm, ep_hbm, cnt_hbm, agg_hbm,
             acc, pb0, pb1, ia0, ib0, ia1, ib1, rb0, rb1, cvec,
             sp0, sp1, sg0, sg1):
        wid = lax.axis_index("s") * NC + lax.axis_index("c")
        lo = wid * R
        ebase = wid * EMAX
        ninf = jnp.full((L,), -jnp.inf, jnp.float32)
        pbs = (pb0, pb1)
        ias = (ia0, ia1)
        ibs = (ib0, ib1)
        rbs = (rb0, rb1)
        sps = (sp0, sp1)
        sgs = (sg0, sg1)

        def initr(r, _):
            acc[pl.ds(r * L, L)] = ninf
            return 0

        lax.fori_loop(0, (R + 1) * D // L, initr, 0)

        pltpu.sync_copy(cnt_hbm.at[pl.ds(pl.multiple_of(wid * L, L), L)], cvec)
        n = cvec[...][0]
        nch = n // CC

        def load_start(k, b):
            base = pl.multiple_of(ebase + k * CC, C)
            return pltpu.async_copy(ep_hbm.at[pl.ds(base, CC)], pbs[b], sps[b])

        def drain_load(b):
            pltpu.make_async_copy(ep_hbm.at[pl.ds(0, CC)], pbs[b], sps[b]).wait()

        def build_and_gather(b):
            pb = pbs[b]
            for half, iref in ((0, ias[b]), (1, ibs[b])):
                for g in range(CI // L):
                    v = pb[pl.ds(half * CI + g * L, L)]
                    iref[pl.ds(g * L, L)] = lax.shift_right_logical(v, 9)
            ca = pltpu.async_copy(h_hbm.at[ias[b]], rbs[b].at[pl.ds(0, CI)],
                                  sgs[b])
            cb = pltpu.async_copy(h_hbm.at[ibs[b]], rbs[b].at[pl.ds(CI, CI)],
                                  sgs[b])
            return ca, cb

        def drain_gather(b):
            pltpu.make_async_copy(h_hbm.at[ias[b]], rbs[b].at[pl.ds(0, CI)],
                                  sgs[b]).wait()
            pltpu.make_async_copy(h_hbm.at[ibs[b]], rbs[b].at[pl.ds(CI, CI)],
                                  sgs[b]).wait()

        def accumulate(b):
            pb = pbs[b]
            rb = rbs[b]
            nj = D // L

            def grp(g, _):
                av = (pb[pl.ds(g * L, L)] & (PK - 1)) * D
                addrs = [av[e] for e in range(L)]
                for e in range(L):
                    base = addrs[e]
                    ei = g * L + e
                    rsl = [rb[ei, pl.ds(j * L, L)] for j in range(nj)]
                    asl = [acc[pl.ds(base + j * L, L)] for j in range(nj)]
                    for j in range(nj):
                        acc[pl.ds(base + j * L, L)] = jnp.maximum(asl[j], rsl[j])
                return 0

            lax.fori_loop(0, CC // L, grp, 0)

        @pl.when(nch > 0)
        def _pro0():
            load_start(0, 0)
            drain_load(0)
            build_and_gather(0)

        @pl.when(nch > 1)
        def _pro1():
            load_start(1, 1)

        def pair(p2, _):
            for b in (0, 1):
                k = p2 * 2 + b

                @pl.when(k < nch)
                def _step():
                    @pl.when(k + 1 < nch)
                    def _bnext():
                        drain_load(1 - b)
                        build_and_gather(1 - b)

                    drain_gather(b)
                    accumulate(b)

                    @pl.when(k + 2 < nch)
                    def _anext():
                        load_start(k + 2, b)

            return 0

        lax.fori_loop(0, (nch + 1) // 2, pair, 0)

        def tail(t, _):
            k = nch * CC + t * C

            @pl.when(k < n)
            def _tl():
                base = pl.multiple_of(ebase + k, C)
                pltpu.sync_copy(ep_hbm.at[pl.ds(base, C)], pb0.at[pl.ds(0, C)])
                zer = jnp.zeros((L,), jnp.int32)
                for g in range(CI // L):
                    if g < C // L:
                        v = pb0[pl.ds(g * L, L)]
                        ia0[pl.ds(g * L, L)] = lax.shift_right_logical(v, 9)
                    else:
                        ia0[pl.ds(g * L, L)] = zer
                pltpu.async_copy(h_hbm.at[ia0], rb0.at[pl.ds(0, CI)],
                                 sg0).wait()

                def grp(g, _):
                    av = (pb0[pl.ds(g * L, L)] & (PK - 1)) * D
                    addrs = [av[e] for e in range(L)]
                    for e in range(L):
                        bs = addrs[e]
                        ei = g * L + e
                        rsl = [rb0[ei, pl.ds(j * L, L)] for j in range(D // L)]
                        asl = [acc[pl.ds(bs + j * L, L)] for j in range(D // L)]
                        for j in range(D // L):
                            acc[pl.ds(bs + j * L, L)] = jnp.maximum(
                                asl[j], rsl[j])
                    return 0

                lax.fori_loop(0, C // L, grp, 0)

            return 0

        lax.fori_loop(0, CC // C, tail, 0)

        def fixr(r, _):
            sl = pl.ds(r * L, L)
            v = acc[sl]
            acc[sl] = jnp.where(v == -jnp.inf, 0.0, v)
            return 0

        lax.fori_loop(0, R * D // L, fixr, 0)
        pltpu.sync_copy(acc.at[pl.ds(0, R * D)],
                        agg_hbm.at[pl.ds(pl.multiple_of(lo * D, D), R * D)])

    return kern(h, ep, cnt)


def _tc_layer(agg, h, wl, wr, bias, last):
    M = 1000
    G = NN // M

    def body(a_ref, h_ref, wl_ref, wr_ref, b_ref, o_ref):
        acc = jnp.dot(a_ref[...], wl_ref[...], preferred_element_type=jnp.float32)
        acc = acc + jnp.dot(h_ref[...], wr_ref[...], preferred_element_type=jnp.float32)
        acc = acc + b_ref[...]
        if last:
            o_ref[...] = jnp.tanh(acc) * 0.5
        else:
            o_ref[...] = jnp.where(acc >= 0, acc, 0.02 * acc)

    return pl.pallas_call(
        body,
        grid=(G,),
        in_specs=[
            pl.BlockSpec((M, D), lambda i: (i, 0)),
            pl.BlockSpec((M, D), lambda i: (i, 0)),
            pl.BlockSpec((D, D), lambda i: (0, 0)),
            pl.BlockSpec((D, D), lambda i: (0, 0)),
            pl.BlockSpec((1, D), lambda i: (0, 0)),
        ],
        out_specs=pl.BlockSpec((M, D), lambda i: (i, 0)),
        out_shape=jax.ShapeDtypeStruct((NN, D), jnp.float32),
    )(agg, h, wl, wr, bias)


def kernel(x, edge_index, Wl, Wr, b):
    src = edge_index[0].astype(jnp.int32)
    dst = edge_index[1].astype(jnp.int32)
    pad = NCHKS * CHK - NE
    src = jnp.concatenate([src, jnp.zeros((pad,), jnp.int32)])
    dst = jnp.concatenate([dst, jnp.full((pad,), NN_PAD, jnp.int32)])
    ep, cnt = _partition_edges(src, dst)
    b2 = b.reshape(NLAYERS, 1, D)
    h = x
    for i in range(NLAYERS):
        agg = _segmax(h, ep, cnt).reshape(NN_PAD, D)[:NN]
        h = _tc_layer(agg, h, Wl[i], Wr[i], b2[i], last=(i == NLAYERS - 1))
    return h
